# baseline (speedup 1.0000x reference)
import functools

import jax
import jax.numpy as jnp
from jax import lax
from jax.experimental import pallas as pl
from jax.experimental.pallas import tpu as pltpu
from jax.experimental.pallas import tpu_sc as plsc

N = 10000
E = 160000
D = 768
H = 12
DH = 64

L = 16
NC = 2
NS = 16
NW = NC * NS
C = 32
NREG = D // L

BM = 256
NP = 10240

EPAD = 160


def _k1_body(x_ref, w_ref, al_ref, ar_ref, feat_ref, el_ref, er_ref):
    f = jnp.dot(x_ref[...], w_ref[...], preferred_element_type=jnp.float32,
                 precision=lax.Precision.HIGHEST)
    feat_ref[...] = f
    el_ref[...] = jnp.dot(f, al_ref[...], preferred_element_type=jnp.float32,
                 precision=lax.Precision.HIGHEST)
    er_ref[...] = jnp.dot(f, ar_ref[...], preferred_element_type=jnp.float32,
                 precision=lax.Precision.HIGHEST)


def _k1(x_p, al, ar, w):
    grid = (NP // BM,)
    return pl.pallas_call(
        _k1_body,
        grid=grid,
        in_specs=[
            pl.BlockSpec((BM, D), lambda i: (i, 0)),
            pl.BlockSpec((D, D), lambda i: (0, 0)),
            pl.BlockSpec((D, 128), lambda i: (0, 0)),
            pl.BlockSpec((D, 128), lambda i: (0, 0)),
        ],
        out_specs=[
            pl.BlockSpec((BM, D), lambda i: (i, 0)),
            pl.BlockSpec((BM, 128), lambda i: (i, 0)),
            pl.BlockSpec((BM, 128), lambda i: (i, 0)),
        ],
        out_shape=[
            jax.ShapeDtypeStruct((NP, D), jnp.float32),
            jax.ShapeDtypeStruct((NP, 128), jnp.float32),
            jax.ShapeDtypeStruct((NP, 128), jnp.float32),
        ],
    )(x_p, w, al, ar)


def _lane_iota():
    return lax.iota(jnp.int32, L)


def _bcast_i32(s):
    return lax.broadcast(s, (L,))


def _scal(ref, i):
    return ref[pl.ds(i, L)][0]


def _k2_body(feat_hbm, el_hbm, er_hbm, srcs_hbm, dsts_hbm, eb_hbm, out_hbm,
             srcv0, dstv0, featv0, elv0, erv0,
             srcv1, dstv1, featv1, elv1, erv1,
             eebuf, invbuf, accv, rowv, zerov, boundsv,
             isem0, isem1, gsem0, gsem1):
    wid = lax.axis_index("c") * NS + lax.axis_index("s")
    n0 = (wid * N) // NW
    n1 = ((wid + 1) * N) // NW

    bufs = ((srcv0, dstv0, featv0, elv0, erv0, isem0, gsem0),
            (srcv1, dstv1, featv1, elv1, erv1, isem1, gsem1))

    pltpu.sync_copy(eb_hbm, boundsv.at[pl.ds(0, 3 * L)])
    e_lo = _scal(boundsv, wid)
    e_hi = _scal(boundsv, wid + 1)
    estart = (e_lo // 8) * 8
    npair = (e_hi - estart + 2 * C - 1) // (2 * C)

    for j in range(NREG):
        zerov[pl.ds(j * L, L)] = jnp.zeros((L,), jnp.float32)
        accv[pl.ds(j * L, L)] = jnp.zeros((L,), jnp.float32)

    def finalize(d_node, acc_ee):
        invbuf[...] = 1.0 / (acc_ee + 1e-9)
        for h in range(H):
            ah = plsc.load_gather(invbuf, [_bcast_i32(h)])
            for q in range(4):
                sl = pl.ds((h * 4 + q) * L, L)
                rowv[sl] = accv[sl] * ah
        pltpu.sync_copy(rowv, out_hbm.at[d_node])

    def zero_fill(lo, hi):
        def zf(g, carry):
            pltpu.sync_copy(zerov, out_hbm.at[g])
            return carry
        lax.fori_loop(lo, hi, zf, 0)

    def issue(k, buf):
        srcv, dstv, featv, elv, erv, isem, gsem = buf
        base = estart + k * C
        di0 = pltpu.async_copy(srcs_hbm.at[pl.ds(base, C)], srcv, isem)
        di1 = pltpu.async_copy(dsts_hbm.at[pl.ds(base, C)],
                               dstv.at[pl.ds(0, C)], isem)
        di0.wait()
        di1.wait()
        pltpu.async_copy(feat_hbm.at[srcv], featv, gsem)
        pltpu.async_copy(el_hbm.at[srcv], elv, gsem)
        pltpu.async_copy(er_hbm.at[dstv.at[pl.ds(0, C)]], erv, gsem)

    def drain(buf):
        srcv, dstv, featv, elv, erv, isem, gsem = buf
        pltpu.make_async_copy(feat_hbm.at[srcv], featv, gsem).wait()
        pltpu.make_async_copy(el_hbm.at[srcv], elv, gsem).wait()
        pltpu.make_async_copy(er_hbm.at[dstv.at[pl.ds(0, C)]], erv,
                              gsem).wait()

    def process(buf, carry):
        srcv, dstv, featv, elv, erv, isem, gsem = buf

        def edge_step(e, ecarry):
            cur, nextz, acc_ee = ecarry
            d = _scal(dstv, e)
            valid = (d >= n0) & (d < n1)
            is_new = valid & (d != cur)

            @pl.when(is_new)
            def _():
                @pl.when(cur >= 0)
                def _():
                    finalize(cur, acc_ee)
                zero_fill(nextz, d)
                for j in range(NREG):
                    accv[pl.ds(j * L, L)] = jnp.zeros((L,), jnp.float32)

            lr = elv[e, pl.ds(0, L)] + erv[e, pl.ds(0, L)]
            lr = jnp.where(lr > 0.0, lr, 0.2 * lr)
            ee_row = jnp.where(lax.broadcast(valid, (L,)), jnp.exp(lr), 0.0)
            eebuf[...] = ee_row
            for h in range(H):
                a = plsc.load_gather(eebuf, [_bcast_i32(h)])
                for q in range(4):
                    sl = pl.ds((h * 4 + q) * L, L)
                    plsc.addupdate(accv.at[sl], featv[e, sl] * a)

            acc_ee = jnp.where(lax.broadcast(is_new, (L,)), 0.0, acc_ee) + ee_row
            cur = jnp.where(is_new, d, cur)
            nextz = jnp.where(is_new, d + 1, nextz)
            return cur, nextz, acc_ee

        return lax.fori_loop(0, C, edge_step, carry, unroll=4)

    issue(0, bufs[0])
    issue(1, bufs[1])

    def pair_step(g, carry):
        for b in range(2):
            k = 2 * g + b
            drain(bufs[b])
            carry = process(bufs[b], carry)
            issue(k + 2, bufs[b])
        return carry

    init = (jnp.int32(-1), n0, jnp.zeros((L,), jnp.float32))
    cur, nextz, acc_ee = lax.fori_loop(0, npair, pair_step, init)
    drain(bufs[0])
    drain(bufs[1])

    @pl.when(cur >= 0)
    def _():
        finalize(cur, acc_ee)
    zero_fill(nextz, n1)


def _k2(feat_p, el_t, er_t, srcs_p, dsts_p, eb_p):
    mesh = plsc.VectorSubcoreMesh(
        core_axis_name="c", subcore_axis_name="s", num_cores=NC,
        num_subcores=NS)
    kern = pl.kernel(
        _k2_body,
        out_type=jax.ShapeDtypeStruct((N, D), jnp.float32),
        mesh=mesh,
        compiler_params=pltpu.CompilerParams(needs_layout_passes=False),
        scratch_types=(
            [pltpu.VMEM((C,), jnp.int32),
             pltpu.VMEM((C + L,), jnp.int32),
             pltpu.VMEM((C, D), jnp.float32),
             pltpu.VMEM((C, 128), jnp.float32),
             pltpu.VMEM((C, 128), jnp.float32)]
            * 2
            + [
                pltpu.VMEM((L,), jnp.float32),
                pltpu.VMEM((L,), jnp.float32),
                pltpu.VMEM((D,), jnp.float32),
                pltpu.VMEM((D,), jnp.float32),
                pltpu.VMEM((D,), jnp.float32),
                pltpu.VMEM((4 * L,), jnp.int32),
                pltpu.SemaphoreType.DMA,
                pltpu.SemaphoreType.DMA,
                pltpu.SemaphoreType.DMA,
                pltpu.SemaphoreType.DMA,
            ]),
    )
    return kern(feat_p, el_t, er_t, srcs_p, dsts_p, eb_p)


def _k3a_body(rst_ref, x_ref, w_ref, b_ref, out_ref, ps_ref, pq_ref):
    t = rst_ref[...] + x_ref[...]
    o = jnp.dot(t, w_ref[...], preferred_element_type=jnp.float32,
                 precision=lax.Precision.HIGHEST) + b_ref[...]
    out_ref[...] = o
    ps_ref[...] = jnp.sum(o, axis=0, keepdims=True)[None]
    pq_ref[...] = jnp.sum(o * o, axis=0, keepdims=True)[None]


def _k3a(rst_p, x_p, w, b):
    grid = (NP // BM,)
    return pl.pallas_call(
        _k3a_body,
        grid=grid,
        in_specs=[
            pl.BlockSpec((BM, D), lambda i: (i, 0)),
            pl.BlockSpec((BM, D), lambda i: (i, 0)),
            pl.BlockSpec((D, D), lambda i: (0, 0)),
            pl.BlockSpec((1, D), lambda i: (0, 0)),
        ],
        out_specs=[
            pl.BlockSpec((BM, D), lambda i: (i, 0)),
            pl.BlockSpec((1, 1, D), lambda i: (i, 0, 0)),
            pl.BlockSpec((1, 1, D), lambda i: (i, 0, 0)),
        ],
        out_shape=[
            jax.ShapeDtypeStruct((NP, D), jnp.float32),
            jax.ShapeDtypeStruct((NP // BM, 1, D), jnp.float32),
            jax.ShapeDtypeStruct((NP // BM, 1, D), jnp.float32),
        ],
    )(rst_p, x_p, w, b)


def _k3b_body(o_ref, sc_ref, sh_ref, w1_ref, b1_ref, w2_ref, b2_ref,
              w3_ref, b3_ref, out_ref):
    h = jnp.maximum(o_ref[...] * sc_ref[...] + sh_ref[...], 0.0)
    h1 = jnp.dot(h, w1_ref[...], preferred_element_type=jnp.float32,
                 precision=lax.Precision.HIGHEST)
    h1 = jnp.maximum(h1 + b1_ref[...], 0.0)
    h2 = jnp.dot(h1, w2_ref[...], preferred_element_type=jnp.float32,
                 precision=lax.Precision.HIGHEST)
    h2 = jnp.maximum(h2 + b2_ref[...], 0.0)
    out_ref[...] = (
        jnp.dot(h2, w3_ref[...], preferred_element_type=jnp.float32,
                 precision=lax.Precision.HIGHEST)
        + b3_ref[...])


def _k3b(out1, scale, shift, w1, b1, w2p, b2p, w3p, b3p):
    grid = (NP // BM,)
    return pl.pallas_call(
        _k3b_body,
        grid=grid,
        in_specs=[
            pl.BlockSpec((BM, D), lambda i: (i, 0)),
            pl.BlockSpec((1, D), lambda i: (0, 0)),
            pl.BlockSpec((1, D), lambda i: (0, 0)),
            pl.BlockSpec((D, 512), lambda i: (0, 0)),
            pl.BlockSpec((1, 512), lambda i: (0, 0)),
            pl.BlockSpec((512, 128), lambda i: (0, 0)),
            pl.BlockSpec((1, 128), lambda i: (0, 0)),
            pl.BlockSpec((128, 128), lambda i: (0, 0)),
            pl.BlockSpec((1, 128), lambda i: (0, 0)),
        ],
        out_specs=pl.BlockSpec((BM, 128), lambda i: (i, 0)),
        out_shape=jax.ShapeDtypeStruct((NP, 128), jnp.float32),
    )(out1, scale, shift, w1, b1, w2p, b2p, w3p, b3p)


def kernel(x, edge_index, W_gat, attn_l, attn_r, fc_out_W, fc_out_b,
           bn_gamma, bn_beta, W1, b1, W2, b2, W3, b3):
    src = edge_index[0]
    dst = edge_index[1]

    perm = jnp.argsort(dst)
    dsts = dst[perm]
    srcs = src[perm]
    node_bounds = ((jnp.arange(NW + 1, dtype=jnp.int32) * N) // NW)
    eb = jnp.searchsorted(dsts, node_bounds, side="left").astype(jnp.int32)
    eb_p = jnp.pad(eb, (0, 3 * L - (NW + 1)), constant_values=E)
    srcs_p = jnp.pad(srcs, (0, EPAD))
    dsts_p = jnp.pad(dsts, (0, EPAD), constant_values=N)

    rows = jnp.arange(D, dtype=jnp.int32)[:, None] // DH
    cols = jnp.arange(128, dtype=jnp.int32)[None, :]
    blockmask = rows == cols
    A_l = jnp.where(blockmask, attn_l.reshape(D)[:, None], 0.0)
    A_r = jnp.where(blockmask, attn_r.reshape(D)[:, None], 0.0)

    x_p = jnp.pad(x, ((0, NP - N), (0, 0)))
    feat_p, el_t, er_t = _k1(x_p, A_l, A_r, W_gat)

    rst = _k2(feat_p, el_t, er_t, srcs_p, dsts_p, eb_p)

    rst_p = jnp.pad(rst, ((0, NP - N), (0, 0)))
    out1, psum, psq = _k3a(rst_p, x_p, fc_out_W, fc_out_b[None, :])

    npad = NP - N
    tot = jnp.sum(psum, axis=(0, 1)) - npad * fc_out_b
    totsq = jnp.sum(psq, axis=(0, 1)) - npad * fc_out_b * fc_out_b
    mean = tot / N
    var = totsq / N - mean * mean
    rstd = 1.0 / jnp.sqrt(var + 1e-5)
    scale = bn_gamma * rstd
    shift = bn_beta - mean * scale

    w2p = jnp.pad(W2, ((0, 0), (0, 128 - 56)))
    b2p = jnp.pad(b2, (0, 128 - 56))
    w3p = jnp.pad(W3, ((0, 128 - 56), (0, 128 - 2)))
    b3p = jnp.pad(b3, (0, 128 - 2))

    logits_p = _k3b(out1, scale[None, :], shift[None, :], W1, b1[None, :],
                    w2p, b2p[None, :], w3p, b3p[None, :])
    return logits_p[:N, :2]


# speedup vs baseline: 1.5564x; 1.5564x over previous
import functools

import jax
import jax.numpy as jnp
from jax import lax
from jax.experimental import pallas as pl
from jax.experimental.pallas import tpu as pltpu
from jax.experimental.pallas import tpu_sc as plsc

N = 10000
E = 160000
D = 768
H = 12
DH = 64

L = 16
NC = 2
NS = 16
NW = NC * NS
C = 32
NREG = D // L

BM = 256
NP = 10240

EPAD = 160


def _bdot(a, b):
    return jnp.dot(a.astype(jnp.bfloat16), b.astype(jnp.bfloat16),
                   preferred_element_type=jnp.float32)


def _k1_body(x_ref, w_ref, al_ref, ar_ref, feat_ref, el_ref, er_ref):
    f = _bdot(x_ref[...], w_ref[...])
    feat_ref[...] = f
    el_ref[...] = _bdot(f, al_ref[...])
    er_ref[...] = _bdot(f, ar_ref[...])


def _k1(x_p, al, ar, w):
    grid = (NP // BM,)
    return pl.pallas_call(
        _k1_body,
        grid=grid,
        in_specs=[
            pl.BlockSpec((BM, D), lambda i: (i, 0)),
            pl.BlockSpec((D, D), lambda i: (0, 0)),
            pl.BlockSpec((D, 128), lambda i: (0, 0)),
            pl.BlockSpec((D, 128), lambda i: (0, 0)),
        ],
        out_specs=[
            pl.BlockSpec((BM, D), lambda i: (i, 0)),
            pl.BlockSpec((BM, 128), lambda i: (i, 0)),
            pl.BlockSpec((BM, 128), lambda i: (i, 0)),
        ],
        out_shape=[
            jax.ShapeDtypeStruct((NP, D), jnp.float32),
            jax.ShapeDtypeStruct((NP, 128), jnp.float32),
            jax.ShapeDtypeStruct((NP, 128), jnp.float32),
        ],
    )(x_p, w, al, ar)


def _lane_iota():
    return lax.iota(jnp.int32, L)


def _bcast_i32(s):
    return lax.broadcast(s, (L,))


def _scal(ref, i):
    return ref[pl.ds(i, L)][0]


_LOG2E = 1.4426950408889634
_LN2_HI = 0.6931471824645996
_LN2_LO = -1.904654323148236e-09


def _exp16(x):
    half = jnp.where(x >= 0.0, 0.5, -0.5)
    k = (x * _LOG2E + half).astype(jnp.int32)
    kf = k.astype(jnp.float32)
    r = (x - kf * _LN2_HI) - kf * _LN2_LO
    p = 1.0 / 720.0 + r * (1.0 / 5040.0)
    for c in (1.0 / 120.0, 1.0 / 24.0, 1.0 / 6.0, 0.5, 1.0, 1.0):
        p = c + r * p
    kc = jnp.clip(k, -126, 127)
    s = plsc.bitcast(lax.shift_left(kc + 127, jnp.full((L,), 23, jnp.int32)),
                     jnp.float32)
    return p * s


def _k2_body(feat_hbm, el_hbm, er_hbm, srcs_hbm, dsts_hbm, eb_hbm, out_hbm,
             srcv0, dstv0, featv0, elv0, erv0,
             srcv1, dstv1, featv1, elv1, erv1,
             eebuf, invbuf, accv, rowv, zerov, boundsv,
             isem0, isem1, gsem0, gsem1):
    wid = lax.axis_index("c") * NS + lax.axis_index("s")
    n0 = (wid * N) // NW
    n1 = ((wid + 1) * N) // NW

    bufs = ((srcv0, dstv0, featv0, elv0, erv0, isem0, gsem0),
            (srcv1, dstv1, featv1, elv1, erv1, isem1, gsem1))

    pltpu.sync_copy(eb_hbm, boundsv.at[pl.ds(0, 3 * L)])
    e_lo = _scal(boundsv, wid)
    e_hi = _scal(boundsv, wid + 1)
    estart = (e_lo // 8) * 8
    npair = (e_hi - estart + 2 * C - 1) // (2 * C)

    for j in range(NREG):
        zerov[pl.ds(j * L, L)] = jnp.zeros((L,), jnp.float32)
        accv[pl.ds(j * L, L)] = jnp.zeros((L,), jnp.float32)

    def finalize(d_node, acc_ee):
        invbuf[...] = 1.0 / (acc_ee + 1e-9)
        for h in range(H):
            ah = plsc.load_gather(invbuf, [_bcast_i32(h)])
            for q in range(4):
                sl = pl.ds((h * 4 + q) * L, L)
                rowv[sl] = accv[sl] * ah
        pltpu.sync_copy(rowv, out_hbm.at[d_node])

    def zero_fill(lo, hi):
        def zf(g, carry):
            pltpu.sync_copy(zerov, out_hbm.at[g])
            return carry
        lax.fori_loop(lo, hi, zf, 0)

    def issue(k, buf):
        srcv, dstv, featv, elv, erv, isem, gsem = buf
        base = estart + k * C
        di0 = pltpu.async_copy(srcs_hbm.at[pl.ds(base, C)], srcv, isem)
        di1 = pltpu.async_copy(dsts_hbm.at[pl.ds(base, C)],
                               dstv.at[pl.ds(0, C)], isem)
        di0.wait()
        di1.wait()
        pltpu.async_copy(feat_hbm.at[srcv], featv, gsem)
        pltpu.async_copy(el_hbm.at[srcv], elv, gsem)
        pltpu.async_copy(er_hbm.at[dstv.at[pl.ds(0, C)]], erv, gsem)

    def drain(buf):
        srcv, dstv, featv, elv, erv, isem, gsem = buf
        pltpu.make_async_copy(feat_hbm.at[srcv], featv, gsem).wait()
        pltpu.make_async_copy(el_hbm.at[srcv], elv, gsem).wait()
        pltpu.make_async_copy(er_hbm.at[dstv.at[pl.ds(0, C)]], erv,
                              gsem).wait()

    def process(buf, carry):
        srcv, dstv, featv, elv, erv, isem, gsem = buf

        def edge_step(e, ecarry):
            cur, nextz, acc_ee = ecarry
            d = _scal(dstv, e)
            valid = (d >= n0) & (d < n1)
            is_new = valid & (d != cur)

            @pl.when(is_new)
            def _():
                @pl.when(cur >= 0)
                def _():
                    finalize(cur, acc_ee)
                zero_fill(nextz, d)
                for j in range(NREG):
                    accv[pl.ds(j * L, L)] = jnp.zeros((L,), jnp.float32)

            lr = elv[e, pl.ds(0, L)] + erv[e, pl.ds(0, L)]
            lr = jnp.where(lr > 0.0, lr, 0.2 * lr)
            ee_row = jnp.where(lax.broadcast(valid, (L,)), jnp.exp(lr), 0.0)
            eebuf[...] = ee_row
            for h in range(H):
                a = plsc.load_gather(eebuf, [_bcast_i32(h)])
                for q in range(4):
                    sl = pl.ds((h * 4 + q) * L, L)
                    plsc.addupdate(accv.at[sl], featv[e, sl] * a)

            acc_ee = jnp.where(lax.broadcast(is_new, (L,)), 0.0, acc_ee) + ee_row
            cur = jnp.where(is_new, d, cur)
            nextz = jnp.where(is_new, d + 1, nextz)
            return cur, nextz, acc_ee

        return lax.fori_loop(0, C, edge_step, carry)

    issue(0, bufs[0])
    issue(1, bufs[1])

    def pair_step(g, carry):
        for b in range(2):
            k = 2 * g + b
            drain(bufs[b])
            carry = process(bufs[b], carry)
            issue(k + 2, bufs[b])
        return carry

    init = (jnp.int32(-1), n0, jnp.zeros((L,), jnp.float32))
    cur, nextz, acc_ee = lax.fori_loop(0, npair, pair_step, init)
    drain(bufs[0])
    drain(bufs[1])

    @pl.when(cur >= 0)
    def _():
        finalize(cur, acc_ee)
    zero_fill(nextz, n1)


def _k2(feat_p, el_t, er_t, srcs_p, dsts_p, eb_p):
    mesh = plsc.VectorSubcoreMesh(
        core_axis_name="c", subcore_axis_name="s", num_cores=NC,
        num_subcores=NS)
    kern = pl.kernel(
        _k2_body,
        out_type=jax.ShapeDtypeStruct((N, D), jnp.float32),
        mesh=mesh,
        compiler_params=pltpu.CompilerParams(needs_layout_passes=False),
        scratch_types=(
            [pltpu.VMEM((C,), jnp.int32),
             pltpu.VMEM((C + L,), jnp.int32),
             pltpu.VMEM((C, D), jnp.float32),
             pltpu.VMEM((C, 128), jnp.float32),
             pltpu.VMEM((C, 128), jnp.float32)]
            * 2
            + [
                pltpu.VMEM((L,), jnp.float32),
                pltpu.VMEM((L,), jnp.float32),
                pltpu.VMEM((D,), jnp.float32),
                pltpu.VMEM((D,), jnp.float32),
                pltpu.VMEM((D,), jnp.float32),
                pltpu.VMEM((4 * L,), jnp.int32),
                pltpu.SemaphoreType.DMA,
                pltpu.SemaphoreType.DMA,
                pltpu.SemaphoreType.DMA,
                pltpu.SemaphoreType.DMA,
            ]),
    )
    return kern(feat_p, el_t, er_t, srcs_p, dsts_p, eb_p)


def _k3a_body(rst_ref, x_ref, w_ref, b_ref, out_ref, ps_ref, pq_ref):
    t = rst_ref[...] + x_ref[...]
    o = _bdot(t, w_ref[...]) + b_ref[...]
    out_ref[...] = o
    ps_ref[...] = jnp.sum(o, axis=0, keepdims=True)[None]
    pq_ref[...] = jnp.sum(o * o, axis=0, keepdims=True)[None]


def _k3a(rst_p, x_p, w, b):
    grid = (NP // BM,)
    return pl.pallas_call(
        _k3a_body,
        grid=grid,
        in_specs=[
            pl.BlockSpec((BM, D), lambda i: (i, 0)),
            pl.BlockSpec((BM, D), lambda i: (i, 0)),
            pl.BlockSpec((D, D), lambda i: (0, 0)),
            pl.BlockSpec((1, D), lambda i: (0, 0)),
        ],
        out_specs=[
            pl.BlockSpec((BM, D), lambda i: (i, 0)),
            pl.BlockSpec((1, 1, D), lambda i: (i, 0, 0)),
            pl.BlockSpec((1, 1, D), lambda i: (i, 0, 0)),
        ],
        out_shape=[
            jax.ShapeDtypeStruct((NP, D), jnp.float32),
            jax.ShapeDtypeStruct((NP // BM, 1, D), jnp.float32),
            jax.ShapeDtypeStruct((NP // BM, 1, D), jnp.float32),
        ],
    )(rst_p, x_p, w, b)


def _k3b_body(o_ref, sc_ref, sh_ref, w1_ref, b1_ref, w2_ref, b2_ref,
              w3_ref, b3_ref, out_ref):
    h = jnp.maximum(o_ref[...] * sc_ref[...] + sh_ref[...], 0.0)
    h1 = _bdot(h, w1_ref[...])
    h1 = jnp.maximum(h1 + b1_ref[...], 0.0)
    h2 = _bdot(h1, w2_ref[...])
    h2 = jnp.maximum(h2 + b2_ref[...], 0.0)
    out_ref[...] = (
        _bdot(h2, w3_ref[...])
        + b3_ref[...])


def _k3b(out1, scale, shift, w1, b1, w2p, b2p, w3p, b3p):
    grid = (NP // BM,)
    return pl.pallas_call(
        _k3b_body,
        grid=grid,
        in_specs=[
            pl.BlockSpec((BM, D), lambda i: (i, 0)),
            pl.BlockSpec((1, D), lambda i: (0, 0)),
            pl.BlockSpec((1, D), lambda i: (0, 0)),
            pl.BlockSpec((D, 512), lambda i: (0, 0)),
            pl.BlockSpec((1, 512), lambda i: (0, 0)),
            pl.BlockSpec((512, 128), lambda i: (0, 0)),
            pl.BlockSpec((1, 128), lambda i: (0, 0)),
            pl.BlockSpec((128, 128), lambda i: (0, 0)),
            pl.BlockSpec((1, 128), lambda i: (0, 0)),
        ],
        out_specs=pl.BlockSpec((BM, 128), lambda i: (i, 0)),
        out_shape=jax.ShapeDtypeStruct((NP, 128), jnp.float32),
    )(out1, scale, shift, w1, b1, w2p, b2p, w3p, b3p)


def kernel(x, edge_index, W_gat, attn_l, attn_r, fc_out_W, fc_out_b,
           bn_gamma, bn_beta, W1, b1, W2, b2, W3, b3):
    src = edge_index[0]
    dst = edge_index[1]

    perm = jnp.argsort(dst)
    dsts = dst[perm]
    srcs = src[perm]
    node_bounds = ((jnp.arange(NW + 1, dtype=jnp.int32) * N) // NW)
    eb = jnp.searchsorted(dsts, node_bounds, side="left").astype(jnp.int32)
    eb_p = jnp.pad(eb, (0, 3 * L - (NW + 1)), constant_values=E)
    srcs_p = jnp.pad(srcs, (0, EPAD))
    dsts_p = jnp.pad(dsts, (0, EPAD), constant_values=N)

    rows = jnp.arange(D, dtype=jnp.int32)[:, None] // DH
    cols = jnp.arange(128, dtype=jnp.int32)[None, :]
    blockmask = rows == cols
    A_l = jnp.where(blockmask, attn_l.reshape(D)[:, None], 0.0)
    A_r = jnp.where(blockmask, attn_r.reshape(D)[:, None], 0.0)

    x_p = jnp.pad(x, ((0, NP - N), (0, 0)))
    feat_p, el_t, er_t = _k1(x_p, A_l, A_r, W_gat)

    rst = _k2(feat_p, el_t, er_t, srcs_p, dsts_p, eb_p)

    rst_p = jnp.pad(rst, ((0, NP - N), (0, 0)))
    out1, psum, psq = _k3a(rst_p, x_p, fc_out_W, fc_out_b[None, :])

    npad = NP - N
    tot = jnp.sum(psum, axis=(0, 1)) - npad * fc_out_b
    totsq = jnp.sum(psq, axis=(0, 1)) - npad * fc_out_b * fc_out_b
    mean = tot / N
    var = totsq / N - mean * mean
    rstd = 1.0 / jnp.sqrt(var + 1e-5)
    scale = bn_gamma * rstd
    shift = bn_beta - mean * scale

    w2p = jnp.pad(W2, ((0, 0), (0, 128 - 56)))
    b2p = jnp.pad(b2, (0, 128 - 56))
    w3p = jnp.pad(W3, ((0, 128 - 56), (0, 128 - 2)))
    b3p = jnp.pad(b3, (0, 128 - 2))

    logits_p = _k3b(out1, scale[None, :], shift[None, :], W1, b1[None, :],
                    w2p, b2p[None, :], w3p, b3p[None, :])
    return logits_p[:N, :2]


# speedup vs baseline: 1.5684x; 1.0078x over previous
import functools

import jax
import jax.numpy as jnp
from jax import lax
from jax.experimental import pallas as pl
from jax.experimental.pallas import tpu as pltpu
from jax.experimental.pallas import tpu_sc as plsc

N = 10000
E = 160000
D = 768
H = 12
DH = 64

L = 16
NC = 2
NS = 16
NW = NC * NS
C = 48
NREG = D // L

BM = 256
NP = 10240

EPAD = 4 * C + 16


def _bdot(a, b):
    return jnp.dot(a.astype(jnp.bfloat16), b.astype(jnp.bfloat16),
                   preferred_element_type=jnp.float32)


def _k1_body(x_ref, w_ref, al_ref, ar_ref, feat_ref, el_ref, er_ref):
    f = _bdot(x_ref[...], w_ref[...])
    feat_ref[...] = f
    el_ref[...] = _bdot(f, al_ref[...])
    er_ref[...] = _bdot(f, ar_ref[...])


def _k1(x_p, al, ar, w):
    grid = (NP // BM,)
    return pl.pallas_call(
        _k1_body,
        grid=grid,
        in_specs=[
            pl.BlockSpec((BM, D), lambda i: (i, 0)),
            pl.BlockSpec((D, D), lambda i: (0, 0)),
            pl.BlockSpec((D, 128), lambda i: (0, 0)),
            pl.BlockSpec((D, 128), lambda i: (0, 0)),
        ],
        out_specs=[
            pl.BlockSpec((BM, D), lambda i: (i, 0)),
            pl.BlockSpec((BM, 128), lambda i: (i, 0)),
            pl.BlockSpec((BM, 128), lambda i: (i, 0)),
        ],
        out_shape=[
            jax.ShapeDtypeStruct((NP, D), jnp.float32),
            jax.ShapeDtypeStruct((NP, 128), jnp.float32),
            jax.ShapeDtypeStruct((NP, 128), jnp.float32),
        ],
    )(x_p, w, al, ar)


def _lane_iota():
    return lax.iota(jnp.int32, L)


def _bcast_i32(s):
    return lax.broadcast(s, (L,))


def _scal(ref, i):
    return ref[pl.ds(i, L)][0]


_LOG2E = 1.4426950408889634
_LN2_HI = 0.6931471824645996
_LN2_LO = -1.904654323148236e-09


def _exp16(x):
    half = jnp.where(x >= 0.0, 0.5, -0.5)
    k = (x * _LOG2E + half).astype(jnp.int32)
    kf = k.astype(jnp.float32)
    r = (x - kf * _LN2_HI) - kf * _LN2_LO
    p = 1.0 / 720.0 + r * (1.0 / 5040.0)
    for c in (1.0 / 120.0, 1.0 / 24.0, 1.0 / 6.0, 0.5, 1.0, 1.0):
        p = c + r * p
    kc = jnp.clip(k, -126, 127)
    s = plsc.bitcast(lax.shift_left(kc + 127, jnp.full((L,), 23, jnp.int32)),
                     jnp.float32)
    return p * s


def _k2_body(feat_hbm, el_hbm, er_hbm, srcs_hbm, dsts_hbm, eb_hbm, out_hbm,
             srcv0, dstv0, featv0, elv0, erv0,
             srcv1, dstv1, featv1, elv1, erv1,
             eebuf, invbuf, accv, rowv, zerov, boundsv,
             isem0, isem1, gsem0, gsem1):
    wid = lax.axis_index("c") * NS + lax.axis_index("s")
    n0 = (wid * N) // NW
    n1 = ((wid + 1) * N) // NW

    bufs = ((srcv0, dstv0, featv0, elv0, erv0, isem0, gsem0),
            (srcv1, dstv1, featv1, elv1, erv1, isem1, gsem1))

    pltpu.sync_copy(eb_hbm, boundsv.at[pl.ds(0, 3 * L)])
    e_lo = _scal(boundsv, wid)
    e_hi = _scal(boundsv, wid + 1)
    estart = (e_lo // 8) * 8
    npair = (e_hi - estart + 2 * C - 1) // (2 * C)

    for j in range(NREG):
        zerov[pl.ds(j * L, L)] = jnp.zeros((L,), jnp.float32)
        accv[pl.ds(j * L, L)] = jnp.zeros((L,), jnp.float32)

    def finalize(d_node, acc_ee):
        invbuf[...] = 1.0 / (acc_ee + 1e-9)
        for h in range(H):
            ah = plsc.load_gather(invbuf, [_bcast_i32(h)])
            for q in range(4):
                sl = pl.ds((h * 4 + q) * L, L)
                rowv[sl] = accv[sl] * ah
        pltpu.sync_copy(rowv, out_hbm.at[d_node])

    def zero_fill(lo, hi):
        def zf(g, carry):
            pltpu.sync_copy(zerov, out_hbm.at[g])
            return carry
        lax.fori_loop(lo, hi, zf, 0)

    def issue(k, buf):
        srcv, dstv, featv, elv, erv, isem, gsem = buf
        base = estart + k * C
        di0 = pltpu.async_copy(srcs_hbm.at[pl.ds(base, C)], srcv, isem)
        di1 = pltpu.async_copy(dsts_hbm.at[pl.ds(base, C)],
                               dstv.at[pl.ds(0, C)], isem)
        di0.wait()
        di1.wait()
        pltpu.async_copy(feat_hbm.at[srcv], featv, gsem)
        pltpu.async_copy(el_hbm.at[srcv], elv, gsem)
        pltpu.async_copy(er_hbm.at[dstv.at[pl.ds(0, C)]], erv, gsem)

    def drain(buf):
        srcv, dstv, featv, elv, erv, isem, gsem = buf
        pltpu.make_async_copy(feat_hbm.at[srcv], featv, gsem).wait()
        pltpu.make_async_copy(el_hbm.at[srcv], elv, gsem).wait()
        pltpu.make_async_copy(er_hbm.at[dstv.at[pl.ds(0, C)]], erv,
                              gsem).wait()

    def process(buf, carry):
        srcv, dstv, featv, elv, erv, isem, gsem = buf

        def edge_step(e, ecarry):
            cur, nextz, acc_ee = ecarry
            d = _scal(dstv, e)
            valid = (d >= n0) & (d < n1)
            is_new = valid & (d != cur)

            @pl.when(is_new)
            def _():
                @pl.when(cur >= 0)
                def _():
                    finalize(cur, acc_ee)
                zero_fill(nextz, d)
                for j in range(NREG):
                    accv[pl.ds(j * L, L)] = jnp.zeros((L,), jnp.float32)

            lr = elv[e, pl.ds(0, L)] + erv[e, pl.ds(0, L)]
            lr = jnp.where(lr > 0.0, lr, 0.2 * lr)
            ee_row = jnp.where(lax.broadcast(valid, (L,)), jnp.exp(lr), 0.0)
            eebuf[...] = ee_row
            for h in range(H):
                a = plsc.load_gather(eebuf, [_bcast_i32(h)])
                for q in range(4):
                    sl = pl.ds((h * 4 + q) * L, L)
                    plsc.addupdate(accv.at[sl], featv[e, sl] * a)

            acc_ee = jnp.where(lax.broadcast(is_new, (L,)), 0.0, acc_ee) + ee_row
            cur = jnp.where(is_new, d, cur)
            nextz = jnp.where(is_new, d + 1, nextz)
            return cur, nextz, acc_ee

        return lax.fori_loop(0, C, edge_step, carry)

    issue(0, bufs[0])
    issue(1, bufs[1])

    def pair_step(g, carry):
        for b in range(2):
            k = 2 * g + b
            drain(bufs[b])
            carry = process(bufs[b], carry)
            issue(k + 2, bufs[b])
        return carry

    init = (jnp.int32(-1), n0, jnp.zeros((L,), jnp.float32))
    cur, nextz, acc_ee = lax.fori_loop(0, npair, pair_step, init)
    drain(bufs[0])
    drain(bufs[1])

    @pl.when(cur >= 0)
    def _():
        finalize(cur, acc_ee)
    zero_fill(nextz, n1)


def _k2(feat_p, el_t, er_t, srcs_p, dsts_p, eb_p):
    mesh = plsc.VectorSubcoreMesh(
        core_axis_name="c", subcore_axis_name="s", num_cores=NC,
        num_subcores=NS)
    kern = pl.kernel(
        _k2_body,
        out_type=jax.ShapeDtypeStruct((N, D), jnp.float32),
        mesh=mesh,
        compiler_params=pltpu.CompilerParams(needs_layout_passes=False),
        scratch_types=(
            [pltpu.VMEM((C,), jnp.int32),
             pltpu.VMEM((C + L,), jnp.int32),
             pltpu.VMEM((C, D), jnp.float32),
             pltpu.VMEM((C, 128), jnp.float32),
             pltpu.VMEM((C, 128), jnp.float32)]
            * 2
            + [
                pltpu.VMEM((L,), jnp.float32),
                pltpu.VMEM((L,), jnp.float32),
                pltpu.VMEM((D,), jnp.float32),
                pltpu.VMEM((D,), jnp.float32),
                pltpu.VMEM((D,), jnp.float32),
                pltpu.VMEM((4 * L,), jnp.int32),
                pltpu.SemaphoreType.DMA,
                pltpu.SemaphoreType.DMA,
                pltpu.SemaphoreType.DMA,
                pltpu.SemaphoreType.DMA,
            ]),
    )
    return kern(feat_p, el_t, er_t, srcs_p, dsts_p, eb_p)


def _k3a_body(rst_ref, x_ref, w_ref, b_ref, out_ref, ps_ref, pq_ref):
    t = rst_ref[...] + x_ref[...]
    o = _bdot(t, w_ref[...]) + b_ref[...]
    out_ref[...] = o
    ps_ref[...] = jnp.sum(o, axis=0, keepdims=True)[None]
    pq_ref[...] = jnp.sum(o * o, axis=0, keepdims=True)[None]


def _k3a(rst_p, x_p, w, b):
    grid = (NP // BM,)
    return pl.pallas_call(
        _k3a_body,
        grid=grid,
        in_specs=[
            pl.BlockSpec((BM, D), lambda i: (i, 0)),
            pl.BlockSpec((BM, D), lambda i: (i, 0)),
            pl.BlockSpec((D, D), lambda i: (0, 0)),
            pl.BlockSpec((1, D), lambda i: (0, 0)),
        ],
        out_specs=[
            pl.BlockSpec((BM, D), lambda i: (i, 0)),
            pl.BlockSpec((1, 1, D), lambda i: (i, 0, 0)),
            pl.BlockSpec((1, 1, D), lambda i: (i, 0, 0)),
        ],
        out_shape=[
            jax.ShapeDtypeStruct((NP, D), jnp.float32),
            jax.ShapeDtypeStruct((NP // BM, 1, D), jnp.float32),
            jax.ShapeDtypeStruct((NP // BM, 1, D), jnp.float32),
        ],
    )(rst_p, x_p, w, b)


def _k3b_body(o_ref, sc_ref, sh_ref, w1_ref, b1_ref, w2_ref, b2_ref,
              w3_ref, b3_ref, out_ref):
    h = jnp.maximum(o_ref[...] * sc_ref[...] + sh_ref[...], 0.0)
    h1 = _bdot(h, w1_ref[...])
    h1 = jnp.maximum(h1 + b1_ref[...], 0.0)
    h2 = _bdot(h1, w2_ref[...])
    h2 = jnp.maximum(h2 + b2_ref[...], 0.0)
    out_ref[...] = (
        _bdot(h2, w3_ref[...])
        + b3_ref[...])


def _k3b(out1, scale, shift, w1, b1, w2p, b2p, w3p, b3p):
    grid = (NP // BM,)
    return pl.pallas_call(
        _k3b_body,
        grid=grid,
        in_specs=[
            pl.BlockSpec((BM, D), lambda i: (i, 0)),
            pl.BlockSpec((1, D), lambda i: (0, 0)),
            pl.BlockSpec((1, D), lambda i: (0, 0)),
            pl.BlockSpec((D, 512), lambda i: (0, 0)),
            pl.BlockSpec((1, 512), lambda i: (0, 0)),
            pl.BlockSpec((512, 128), lambda i: (0, 0)),
            pl.BlockSpec((1, 128), lambda i: (0, 0)),
            pl.BlockSpec((128, 128), lambda i: (0, 0)),
            pl.BlockSpec((1, 128), lambda i: (0, 0)),
        ],
        out_specs=pl.BlockSpec((BM, 128), lambda i: (i, 0)),
        out_shape=jax.ShapeDtypeStruct((NP, 128), jnp.float32),
    )(out1, scale, shift, w1, b1, w2p, b2p, w3p, b3p)


def kernel(x, edge_index, W_gat, attn_l, attn_r, fc_out_W, fc_out_b,
           bn_gamma, bn_beta, W1, b1, W2, b2, W3, b3):
    src = edge_index[0]
    dst = edge_index[1]

    perm = jnp.argsort(dst)
    dsts = dst[perm]
    srcs = src[perm]
    node_bounds = ((jnp.arange(NW + 1, dtype=jnp.int32) * N) // NW)
    eb = jnp.searchsorted(dsts, node_bounds, side="left").astype(jnp.int32)
    eb_p = jnp.pad(eb, (0, 3 * L - (NW + 1)), constant_values=E)
    srcs_p = jnp.pad(srcs, (0, EPAD))
    dsts_p = jnp.pad(dsts, (0, EPAD), constant_values=N)

    rows = jnp.arange(D, dtype=jnp.int32)[:, None] // DH
    cols = jnp.arange(128, dtype=jnp.int32)[None, :]
    blockmask = rows == cols
    A_l = jnp.where(blockmask, attn_l.reshape(D)[:, None], 0.0)
    A_r = jnp.where(blockmask, attn_r.reshape(D)[:, None], 0.0)

    x_p = jnp.pad(x, ((0, NP - N), (0, 0)))
    feat_p, el_t, er_t = _k1(x_p, A_l, A_r, W_gat)

    rst = _k2(feat_p, el_t, er_t, srcs_p, dsts_p, eb_p)

    rst_p = jnp.pad(rst, ((0, NP - N), (0, 0)))
    out1, psum, psq = _k3a(rst_p, x_p, fc_out_W, fc_out_b[None, :])

    npad = NP - N
    tot = jnp.sum(psum, axis=(0, 1)) - npad * fc_out_b
    totsq = jnp.sum(psq, axis=(0, 1)) - npad * fc_out_b * fc_out_b
    mean = tot / N
    var = totsq / N - mean * mean
    rstd = 1.0 / jnp.sqrt(var + 1e-5)
    scale = bn_gamma * rstd
    shift = bn_beta - mean * scale

    w2p = jnp.pad(W2, ((0, 0), (0, 128 - 56)))
    b2p = jnp.pad(b2, (0, 128 - 56))
    w3p = jnp.pad(W3, ((0, 128 - 56), (0, 128 - 2)))
    b3p = jnp.pad(b3, (0, 128 - 2))

    logits_p = _k3b(out1, scale[None, :], shift[None, :], W1, b1[None, :],
                    w2p, b2p[None, :], w3p, b3p[None, :])
    return logits_p[:N, :2]
